# baseline (device time: 22662 ns/iter reference)
import jax
import jax.numpy as jnp
from jax import lax
from jax.experimental import pallas as pl
from jax.experimental.pallas import tpu as pltpu

B, H, D = 8, 8, 64
KLOC = 512
NYZ = 8
KSUB = KLOC // NYZ
NDEV = 16
SCALE = D ** -0.5

_POSITIONS = [(qx, qy, qz, (qx * 2 + qy) * 4 + qz)
              for qx in range(2) for qy in range(2) for qz in range(4)]


def kernel(Q, K, V):
    Q2 = Q.reshape(B, H, D)
    K2 = K.reshape(B, KLOC, H * D)
    V2 = V.reshape(B, KLOC, H * D)

    def body(q_ref, k_ref, v_ref, o_ref, comm, s_sems, r_sems):
        my_x = lax.axis_index("x")
        my_y = lax.axis_index("y")
        my_z = lax.axis_index("z")
        my_lin = (my_x * 2 + my_y) * 4 + my_z
        start = (my_y * 4 + my_z) * KSUB

        bar = pltpu.get_barrier_semaphore()
        for qx, qy, qz, lin_q in _POSITIONS:
            @pl.when(lin_q != my_lin)
            def _(qx=qx, qy=qy, qz=qz):
                pl.semaphore_signal(bar, inc=1, device_id=(qx, qy, qz),
                                    device_id_type=pl.DeviceIdType.MESH)

        rowh = lax.broadcasted_iota(jnp.int32, (H * D, H), 0) // D
        colh = lax.broadcasted_iota(jnp.int32, (H * D, H), 1)
        qmaskT = (rowh == colh).astype(jnp.float32)
        eye3 = (lax.broadcasted_iota(jnp.int32, (H, H, 1), 0)
                == lax.broadcasted_iota(jnp.int32, (H, H, 1), 1)
                ).astype(jnp.float32)

        ms, ls, os_ = [], [], []
        for b in range(B):
            kb = k_ref[b, pl.ds(start, KSUB), :]
            vb = v_ref[b, pl.ds(start, KSUB), :]
            qbT = q_ref[b].T
            qblkT = jnp.concatenate([qbT] * H, axis=0) * qmaskT
            s = lax.dot_general(
                kb, qblkT, (((1,), (0,)), ((), ())),
                preferred_element_type=jnp.float32) * SCALE
            m = jnp.max(s, axis=0, keepdims=True)
            p = jnp.exp(s - m)
            l = jnp.sum(p, axis=0, keepdims=True)
            t = lax.dot_general(
                p, vb, (((0,), (0,)), ((), ())),
                preferred_element_type=jnp.float32)
            ob = jnp.sum(t.reshape(H, H, D) * eye3, axis=0)
            ms.append(m)
            ls.append(l)
            os_.append(ob)

        m_arr = jnp.concatenate(ms, axis=0)
        l_arr = jnp.concatenate(ls, axis=0)
        stat = jnp.concatenate(
            [m_arr, l_arr, jnp.zeros((B, D - 2 * H), jnp.float32)], axis=1)
        msg = jnp.concatenate(
            [jnp.stack(os_, axis=0), stat[None]], axis=0)
        comm[my_lin] = msg.astype(jnp.bfloat16)

        pl.semaphore_wait(bar, NDEV - 1)

        def out_desc(qx, qy, qz, lin_q):
            return pltpu.make_async_remote_copy(
                src_ref=comm.at[my_lin], dst_ref=comm.at[my_lin],
                send_sem=s_sems.at[lin_q], recv_sem=r_sems.at[my_lin],
                device_id=(qx, qy, qz), device_id_type=pl.DeviceIdType.MESH)

        for qx, qy, qz, lin_q in _POSITIONS:
            @pl.when(lin_q != my_lin)
            def _(qx=qx, qy=qy, qz=qz, lin_q=lin_q):
                out_desc(qx, qy, qz, lin_q).start()

        for qx, qy, qz, lin_q in _POSITIONS:
            @pl.when(lin_q != my_lin)
            def _(qx=qx, qy=qy, qz=qz, lin_q=lin_q):
                pltpu.make_async_remote_copy(
                    src_ref=comm.at[lin_q], dst_ref=comm.at[lin_q],
                    send_sem=s_sems.at[lin_q], recv_sem=r_sems.at[lin_q],
                    device_id=(qx, qy, qz),
                    device_id_type=pl.DeviceIdType.MESH).wait_recv()

        call = comm[...].astype(jnp.float32)
        o_all = call[:, :B]
        m_all = call[:, B, :, 0:H]
        l_all = call[:, B, :, H:2 * H]
        m_n = jnp.max(m_all, axis=0)
        w = jnp.exp(m_all - m_n[None])
        l_n = jnp.sum(w * l_all, axis=0)
        o = jnp.sum(w[..., None] * o_all, axis=0) / l_n[..., None]
        o_ref[...] = o[:, None]

        for qx, qy, qz, lin_q in _POSITIONS:
            @pl.when(lin_q != my_lin)
            def _(qx=qx, qy=qy, qz=qz, lin_q=lin_q):
                out_desc(qx, qy, qz, lin_q).wait_send()

    return pl.pallas_call(
        body,
        out_shape=jax.ShapeDtypeStruct((B, 1, H, D), jnp.float32),
        in_specs=[
            pl.BlockSpec(memory_space=pltpu.VMEM),
            pl.BlockSpec(memory_space=pltpu.VMEM),
            pl.BlockSpec(memory_space=pltpu.VMEM),
        ],
        out_specs=pl.BlockSpec(memory_space=pltpu.VMEM),
        scratch_shapes=[
            pltpu.VMEM((NDEV, B + 1, H, D), jnp.bfloat16),
            pltpu.SemaphoreType.DMA((NDEV,)),
            pltpu.SemaphoreType.DMA((NDEV,)),
        ],
        compiler_params=pltpu.CompilerParams(collective_id=0),
    )(Q2, K2, V2)


# device time: 15770 ns/iter; 1.4370x vs baseline; 1.4370x over previous
import jax
import jax.numpy as jnp
from jax import lax
from jax.experimental import pallas as pl
from jax.experimental.pallas import tpu as pltpu

B, H, D = 8, 8, 64
KLOC = 512
NYZ = 8
KSUB = KLOC // NYZ
NDEV = 16
SCALE = D ** -0.5

_POSITIONS = [(qx, qy, qz, (qx * 2 + qy) * 4 + qz)
              for qx in range(2) for qy in range(2) for qz in range(4)]


def kernel(Q, K, V):
    Q2 = Q.reshape(B, H, D)
    K2 = K.reshape(B, KLOC, H * D)
    V2 = V.reshape(B, KLOC, H * D)
    sl = (lax.axis_index("y") * 4 + lax.axis_index("z")) * KSUB
    Ks = lax.dynamic_slice_in_dim(K2, sl, KSUB, axis=1)
    Vs = lax.dynamic_slice_in_dim(V2, sl, KSUB, axis=1)

    def body(q_ref, k_ref, v_ref, o_ref, comm, s_sems, r_sems):
        my_x = lax.axis_index("x")
        my_y = lax.axis_index("y")
        my_z = lax.axis_index("z")
        my_lin = (my_x * 2 + my_y) * 4 + my_z

        bar = pltpu.get_barrier_semaphore()
        for qx, qy, qz, lin_q in _POSITIONS:
            @pl.when(lin_q != my_lin)
            def _(qx=qx, qy=qy, qz=qz):
                pl.semaphore_signal(bar, inc=1, device_id=(qx, qy, qz),
                                    device_id_type=pl.DeviceIdType.MESH)

        rowh = lax.broadcasted_iota(jnp.int32, (H * D, H), 0) // D
        colh = lax.broadcasted_iota(jnp.int32, (H * D, H), 1)
        qmaskT = (rowh == colh).astype(jnp.float32)
        eye3 = (lax.broadcasted_iota(jnp.int32, (H, H, 1), 0)
                == lax.broadcasted_iota(jnp.int32, (H, H, 1), 1)
                ).astype(jnp.float32)

        ms, ls, os_ = [], [], []
        for b in range(B):
            kb = k_ref[b]
            vb = v_ref[b]
            qbT = q_ref[b].T
            qblkT = jnp.concatenate([qbT] * H, axis=0) * qmaskT
            s = lax.dot_general(
                kb, qblkT, (((1,), (0,)), ((), ())),
                preferred_element_type=jnp.float32) * SCALE
            m = jnp.max(s, axis=0, keepdims=True)
            p = jnp.exp(s - m)
            l = jnp.sum(p, axis=0, keepdims=True)
            t = lax.dot_general(
                p, vb, (((0,), (0,)), ((), ())),
                preferred_element_type=jnp.float32)
            ob = jnp.sum(t.reshape(H, H, D) * eye3, axis=0)
            ms.append(m)
            ls.append(l)
            os_.append(ob)

        m_arr = jnp.concatenate(ms, axis=0)
        l_arr = jnp.concatenate(ls, axis=0)
        stat = jnp.concatenate(
            [m_arr, l_arr, jnp.zeros((B, D - 2 * H), jnp.float32)], axis=1)
        msg = jnp.concatenate(
            [jnp.stack(os_, axis=0), stat[None]], axis=0)
        comm[my_lin] = msg.astype(jnp.bfloat16)

        pl.semaphore_wait(bar, NDEV - 1)

        def out_desc(qx, qy, qz, lin_q):
            return pltpu.make_async_remote_copy(
                src_ref=comm.at[my_lin], dst_ref=comm.at[my_lin],
                send_sem=s_sems.at[lin_q], recv_sem=r_sems.at[my_lin],
                device_id=(qx, qy, qz), device_id_type=pl.DeviceIdType.MESH)

        for qx, qy, qz, lin_q in _POSITIONS:
            @pl.when(lin_q != my_lin)
            def _(qx=qx, qy=qy, qz=qz, lin_q=lin_q):
                out_desc(qx, qy, qz, lin_q).start()

        for qx, qy, qz, lin_q in _POSITIONS:
            @pl.when(lin_q != my_lin)
            def _(qx=qx, qy=qy, qz=qz, lin_q=lin_q):
                pltpu.make_async_remote_copy(
                    src_ref=comm.at[lin_q], dst_ref=comm.at[lin_q],
                    send_sem=s_sems.at[lin_q], recv_sem=r_sems.at[lin_q],
                    device_id=(qx, qy, qz),
                    device_id_type=pl.DeviceIdType.MESH).wait_recv()

        call = comm[...].astype(jnp.float32)
        o_all = call[:, :B]
        m_all = call[:, B, :, 0:H]
        l_all = call[:, B, :, H:2 * H]
        m_n = jnp.max(m_all, axis=0)
        w = jnp.exp(m_all - m_n[None])
        l_n = jnp.sum(w * l_all, axis=0)
        o = jnp.sum(w[..., None] * o_all, axis=0) / l_n[..., None]
        o_ref[...] = o[:, None]

        for qx, qy, qz, lin_q in _POSITIONS:
            @pl.when(lin_q != my_lin)
            def _(qx=qx, qy=qy, qz=qz, lin_q=lin_q):
                out_desc(qx, qy, qz, lin_q).wait_send()

    return pl.pallas_call(
        body,
        out_shape=jax.ShapeDtypeStruct((B, 1, H, D), jnp.float32),
        in_specs=[
            pl.BlockSpec(memory_space=pltpu.VMEM),
            pl.BlockSpec(memory_space=pltpu.VMEM),
            pl.BlockSpec(memory_space=pltpu.VMEM),
        ],
        out_specs=pl.BlockSpec(memory_space=pltpu.VMEM),
        scratch_shapes=[
            pltpu.VMEM((NDEV, B + 1, H, D), jnp.bfloat16),
            pltpu.SemaphoreType.DMA((NDEV,)),
            pltpu.SemaphoreType.DMA((NDEV,)),
        ],
        compiler_params=pltpu.CompilerParams(collective_id=0),
    )(Q2, Ks, Vs)


# device time: 15056 ns/iter; 1.5052x vs baseline; 1.0474x over previous
import jax
import jax.numpy as jnp
from jax import lax
from jax.experimental import pallas as pl
from jax.experimental.pallas import tpu as pltpu

B, H, D = 8, 8, 64
KLOC = 512
SCALE = D ** -0.5


def kernel(Q, K, V):
    Q2 = Q.reshape(B, H, D)
    KT = K.reshape(B, KLOC, H * D).transpose(0, 2, 1)
    V2 = V.reshape(B, KLOC, H * D)

    def body(q_ref, k_ref, v_ref, o_ref,
             acc, stats, peer_acc, peer_stats, send_sems, recv_sems):
        my_x = lax.axis_index("x")
        my_y = lax.axis_index("y")
        my_z = lax.axis_index("z")
        partner = (1 - my_x, my_y, my_z)

        barrier_sem = pltpu.get_barrier_semaphore()
        pl.semaphore_signal(barrier_sem, inc=1, device_id=partner,
                            device_id_type=pl.DeviceIdType.MESH)

        colh = lax.broadcasted_iota(jnp.int32, (H, H * D), 1) // D
        rowh = lax.broadcasted_iota(jnp.int32, (H, H * D), 0)
        qmask = (colh == rowh).astype(jnp.float32)
        eye3 = (lax.broadcasted_iota(jnp.int32, (H, H, 1), 0)
                == lax.broadcasted_iota(jnp.int32, (H, H, 1), 1)
                ).astype(jnp.float32)

        ms, ls, os_ = [], [], []
        for b in range(B):
            qb = q_ref[b]
            qblk = jnp.concatenate([qb] * H, axis=1) * qmask
            s = lax.dot_general(
                qblk, k_ref[b], (((1,), (0,)), ((), ())),
                preferred_element_type=jnp.float32) * SCALE
            m = jnp.max(s, axis=1, keepdims=True)
            p = jnp.exp(s - m)
            l = jnp.sum(p, axis=1, keepdims=True)
            t = lax.dot_general(
                p, v_ref[b], (((1,), (0,)), ((), ())),
                preferred_element_type=jnp.float32)
            ob = jnp.sum(t.reshape(H, H, D) * eye3, axis=0)
            ms.append(m.reshape(1, H))
            ls.append(l.reshape(1, H))
            os_.append(ob)
        acc[...] = jnp.stack(os_, axis=0)
        stats[0] = jnp.concatenate(ms, axis=0)
        stats[1] = jnp.concatenate(ls, axis=0)

        pl.semaphore_wait(barrier_sem, 1)

        rdma_o = pltpu.make_async_remote_copy(
            src_ref=acc, dst_ref=peer_acc,
            send_sem=send_sems.at[0], recv_sem=recv_sems.at[0],
            device_id=partner, device_id_type=pl.DeviceIdType.MESH)
        rdma_s = pltpu.make_async_remote_copy(
            src_ref=stats, dst_ref=peer_stats,
            send_sem=send_sems.at[1], recv_sem=recv_sems.at[1],
            device_id=partner, device_id_type=pl.DeviceIdType.MESH)
        rdma_o.start()
        rdma_s.start()
        rdma_o.wait()
        rdma_s.wait()

        m_s, l_s = stats[0], stats[1]
        m_p, l_p = peer_stats[0], peer_stats[1]
        m_n = jnp.maximum(m_s, m_p)
        a_s = jnp.exp(m_s - m_n)
        a_p = jnp.exp(m_p - m_n)
        l_n = a_s * l_s + a_p * l_p
        o = (a_s[:, :, None] * acc[...] + a_p[:, :, None] * peer_acc[...]) \
            / l_n[:, :, None]
        o_ref[...] = o[:, None]

    return pl.pallas_call(
        body,
        out_shape=jax.ShapeDtypeStruct((B, 1, H, D), jnp.float32),
        in_specs=[
            pl.BlockSpec(memory_space=pltpu.VMEM),
            pl.BlockSpec(memory_space=pltpu.VMEM),
            pl.BlockSpec(memory_space=pltpu.VMEM),
        ],
        out_specs=pl.BlockSpec(memory_space=pltpu.VMEM),
        scratch_shapes=[
            pltpu.VMEM((B, H, D), jnp.float32),
            pltpu.VMEM((2, B, H), jnp.float32),
            pltpu.VMEM((B, H, D), jnp.float32),
            pltpu.VMEM((2, B, H), jnp.float32),
            pltpu.SemaphoreType.DMA((2,)),
            pltpu.SemaphoreType.DMA((2,)),
        ],
        compiler_params=pltpu.CompilerParams(collective_id=0),
    )(Q2, KT, V2)
